# Initial kernel scaffold; baseline (speedup 1.0000x reference)
import jax
import jax.numpy as jnp
from jax.experimental import pallas as pl


def kernel(x, coeffs, G_rows, G_cols, G_vals, EW, NS, L_rows, L_cols, L_vals, F_rows, F_cols, F_vals):
    raise NotImplementedError("write your pallas kernel here")


# speedup vs baseline: 105.2512x; 105.2512x over previous
import functools

import jax
import jax.numpy as jnp
from jax import lax
from jax.experimental import pallas as pl
from jax.experimental.pallas import tpu as pltpu
from jax.experimental.pallas import tpu_sc as plsc

NV = 40962
NF = 81920
B = 4
CIN = 32
COUT = 32
LAN = B * CIN
NSEG = LAN // 16

NW = 32
CF = 32
FPT = NF // NW
A_CHUNKS = FPT // CF

CV = 32
VPT = 1312
NV_PAD = NW * VPT
B_CHUNKS = VPT // CV

_mesh = plsc.VectorSubcoreMesh(core_axis_name="c", subcore_axis_name="s")


def _wid():
    return lax.axis_index("s") * 2 + lax.axis_index("c")


@functools.partial(
    pl.kernel,
    mesh=_mesh,
    out_type=[
        jax.ShapeDtypeStruct((NF, LAN), jnp.float32),
        jax.ShapeDtypeStruct((NF, LAN), jnp.float32),
    ],
    scratch_types=[
        pltpu.VMEM((3, 3 * CF), jnp.int32),
        pltpu.VMEM((3, 3 * CF + 16), jnp.float32),
        pltpu.VMEM((3 * CF + 16,), jnp.float32),
        pltpu.VMEM((3 * CF + 16,), jnp.float32),
        pltpu.VMEM((9 * CF, LAN), jnp.float32),
        pltpu.VMEM((CF, LAN), jnp.float32),
        pltpu.VMEM((CF, LAN), jnp.float32),
        pltpu.SemaphoreType.DMA,
    ],
)
def _stage_a(xt, gcols, gvals, ewf, nsf, oew, ons,
             idx_v, gv_v, ew_v, ns_v, rows_v, oe_v, on_v, sem):
    wid = _wid()
    f0 = wid * FPT

    def chunk(ch, carry):
        fb = f0 + ch * CF
        for d in range(3):
            off = 3 * d * NF + 3 * fb
            pltpu.sync_copy(gcols.at[pl.ds(off, 3 * CF)], idx_v.at[d])
            pltpu.sync_copy(gvals.at[pl.ds(off, 3 * CF)],
                            gv_v.at[d, pl.ds(0, 3 * CF)])
        pltpu.sync_copy(ewf.at[pl.ds(3 * fb, 3 * CF)], ew_v.at[pl.ds(0, 3 * CF)])
        pltpu.sync_copy(nsf.at[pl.ds(3 * fb, 3 * CF)], ns_v.at[pl.ds(0, 3 * CF)])
        cps = [
            pltpu.async_copy(xt.at[idx_v.at[d]],
                             rows_v.at[pl.ds(d * 3 * CF, 3 * CF)], sem)
            for d in range(3)
        ]
        for c in cps:
            c.wait()

        def face(fl, inner):
            acc_e = [jnp.zeros((16,), jnp.float32) for _ in range(NSEG)]
            acc_n = [jnp.zeros((16,), jnp.float32) for _ in range(NSEG)]
            ew_vec = ew_v[pl.ds(3 * fl, 16)]
            ns_vec = ns_v[pl.ds(3 * fl, 16)]
            gv_vecs = [gv_v[d, pl.ds(3 * fl, 16)] for d in range(3)]
            for d in range(3):
                we_d = ew_vec[d]
                wn_d = ns_vec[d]
                for j in range(3):
                    g = gv_vecs[d][j]
                    we = we_d * g
                    wn = wn_d * g
                    r = d * 3 * CF + 3 * fl + j
                    for s in range(NSEG):
                        seg = rows_v[r, pl.ds(16 * s, 16)]
                        acc_e[s] = acc_e[s] + we * seg
                        acc_n[s] = acc_n[s] + wn * seg
            for s in range(NSEG):
                oe_v[fl, pl.ds(16 * s, 16)] = acc_e[s]
                on_v[fl, pl.ds(16 * s, 16)] = acc_n[s]
            return inner

        lax.fori_loop(0, CF, face, 0)
        pltpu.sync_copy(oe_v, oew.at[pl.ds(fb, CF)])
        pltpu.sync_copy(on_v, ons.at[pl.ds(fb, CF)])
        return carry

    lax.fori_loop(0, A_CHUNKS, chunk, 0)


@functools.partial(
    pl.kernel,
    mesh=_mesh,
    out_type=[
        jax.ShapeDtypeStruct((NV_PAD, LAN), jnp.float32),
        jax.ShapeDtypeStruct((NV_PAD, LAN), jnp.float32),
        jax.ShapeDtypeStruct((NV_PAD, LAN), jnp.float32),
    ],
    scratch_types=[
        pltpu.VMEM((6 * CV,), jnp.int32),
        pltpu.VMEM((6 * CV + 16,), jnp.float32),
        pltpu.VMEM((7 * CV,), jnp.int32),
        pltpu.VMEM((7 * CV + 16,), jnp.float32),
        pltpu.VMEM((6 * CV, LAN), jnp.float32),
        pltpu.VMEM((6 * CV, LAN), jnp.float32),
        pltpu.VMEM((7 * CV, LAN), jnp.float32),
        pltpu.VMEM((CV, LAN), jnp.float32),
        pltpu.VMEM((CV, LAN), jnp.float32),
        pltpu.VMEM((CV, LAN), jnp.float32),
        pltpu.SemaphoreType.DMA,
    ],
)
def _stage_b(gfe, gfn, xt, fcols, fvals, lcols, lvals,
             ove, ovn, olap,
             fi_v, fv_v, li_v, lv_v, re_v, rn_v, rl_v, oe_v, on_v, ol_v, sem):
    wid = _wid()
    v0 = wid * VPT

    def chunk(ch, carry):
        vb = v0 + ch * CV
        pltpu.sync_copy(fcols.at[pl.ds(6 * vb, 6 * CV)], fi_v)
        pltpu.sync_copy(fvals.at[pl.ds(6 * vb, 6 * CV)], fv_v.at[pl.ds(0, 6 * CV)])
        pltpu.sync_copy(lcols.at[pl.ds(7 * vb, 7 * CV)], li_v)
        pltpu.sync_copy(lvals.at[pl.ds(7 * vb, 7 * CV)], lv_v.at[pl.ds(0, 7 * CV)])
        cps = []
        for h in range(2):
            s96 = pl.ds(h * 96, 96)
            cps.append(pltpu.async_copy(gfe.at[fi_v.at[s96]], re_v.at[s96], sem))
            cps.append(pltpu.async_copy(gfn.at[fi_v.at[s96]], rn_v.at[s96], sem))
            s112 = pl.ds(h * 112, 112)
            cps.append(pltpu.async_copy(xt.at[li_v.at[s112]], rl_v.at[s112], sem))
        for c in cps:
            c.wait()

        def vert(vl, inner):
            acc_e = [jnp.zeros((16,), jnp.float32) for _ in range(NSEG)]
            acc_n = [jnp.zeros((16,), jnp.float32) for _ in range(NSEG)]
            acc_l = [jnp.zeros((16,), jnp.float32) for _ in range(NSEG)]
            fv_vec = fv_v[pl.ds(6 * vl, 16)]
            lv_vec = lv_v[pl.ds(7 * vl, 16)]
            for j in range(6):
                r = 6 * vl + j
                w = fv_vec[j]
                for s in range(NSEG):
                    sl = pl.ds(16 * s, 16)
                    acc_e[s] = acc_e[s] + w * re_v[r, sl]
                    acc_n[s] = acc_n[s] + w * rn_v[r, sl]
            for j in range(7):
                r = 7 * vl + j
                w = lv_vec[j]
                for s in range(NSEG):
                    acc_l[s] = acc_l[s] + w * rl_v[r, pl.ds(16 * s, 16)]
            for s in range(NSEG):
                sl = pl.ds(16 * s, 16)
                oe_v[vl, sl] = acc_e[s]
                on_v[vl, sl] = acc_n[s]
                ol_v[vl, sl] = acc_l[s]
            return inner

        lax.fori_loop(0, CV, vert, 0)
        pltpu.sync_copy(oe_v, ove.at[pl.ds(vb, CV)])
        pltpu.sync_copy(on_v, ovn.at[pl.ds(vb, CV)])
        pltpu.sync_copy(ol_v, olap.at[pl.ds(vb, CV)])
        return carry

    lax.fori_loop(0, B_CHUNKS, chunk, 0)


TR = 512
GRID_D = (NV + TR - 1) // TR


def _stage_d_body(id_r, ew_r, ns_r, lp_r, bd_r, out_r):
    acc = jnp.dot(id_r[...], bd_r[0], preferred_element_type=jnp.float32)
    acc = acc + jnp.dot(ew_r[...], bd_r[1], preferred_element_type=jnp.float32)
    acc = acc + jnp.dot(ns_r[...], bd_r[2], preferred_element_type=jnp.float32)
    acc = acc + jnp.dot(lp_r[...], bd_r[3], preferred_element_type=jnp.float32)
    out_r[...] = acc


def _stage_d(x_t, gve, gvn, lap, bd):
    row_spec = pl.BlockSpec((TR, LAN), lambda i: (i, 0))
    return pl.pallas_call(
        _stage_d_body,
        grid=(GRID_D,),
        in_specs=[row_spec, row_spec, row_spec, row_spec,
                  pl.BlockSpec((4, LAN, B * COUT), lambda i: (0, 0, 0))],
        out_specs=pl.BlockSpec((TR, B * COUT), lambda i: (i, 0)),
        out_shape=jax.ShapeDtypeStruct((NV, B * COUT), jnp.float32),
    )(x_t, gve, gvn, lap, bd)


def kernel(x, coeffs, G_rows, G_cols, G_vals, EW, NS,
           L_rows, L_cols, L_vals, F_rows, F_cols, F_vals):
    x_t = x.transpose(2, 0, 1).reshape(NV, LAN)
    ewf = EW.reshape(-1)
    nsf = NS.reshape(-1)
    pad_f = NV_PAD * 6 - NV * 6
    pad_l = NV_PAD * 7 - NV * 7
    fcols = jnp.concatenate([F_cols, jnp.zeros((pad_f,), jnp.int32)])
    fvals = jnp.concatenate([F_vals, jnp.zeros((pad_f,), jnp.float32)])
    lcols = jnp.concatenate([L_cols, jnp.zeros((pad_l,), jnp.int32)])
    lvals = jnp.concatenate([L_vals, jnp.zeros((pad_l,), jnp.float32)])

    gfe, gfn = _stage_a(x_t, G_cols, G_vals, ewf, nsf)
    gve, gvn, lap = _stage_b(gfe, gfn, x_t, fcols, fvals, lcols, lvals)

    wk = coeffs.transpose(2, 1, 0)
    eye = jnp.eye(B, dtype=jnp.float32)
    bd = jax.vmap(lambda w: jnp.kron(eye, w))(wk)

    out_t = _stage_d(x_t, gve, gvn, lap, bd)
    return out_t.reshape(NV, B, COUT).transpose(1, 2, 0)


# speedup vs baseline: 164.9311x; 1.5670x over previous
import functools

import jax
import jax.numpy as jnp
from jax import lax
from jax.experimental import pallas as pl
from jax.experimental.pallas import tpu as pltpu
from jax.experimental.pallas import tpu_sc as plsc

NV = 40962
NF = 81920
B = 4
CIN = 32
COUT = 32
LAN = B * CIN
NSEG = LAN // 16

NW = 32
CF = 32
FPT = NF // NW
A_CHUNKS = FPT // CF

CV = 16
VPT = 1312
NV_PAD = NW * VPT
B_CHUNKS = VPT // CV

_mesh = plsc.VectorSubcoreMesh(core_axis_name="c", subcore_axis_name="s")


def _wid():
    return lax.axis_index("s") * 2 + lax.axis_index("c")


@functools.partial(
    pl.kernel,
    mesh=_mesh,
    out_type=[
        jax.ShapeDtypeStruct((NF, LAN), jnp.float32),
        jax.ShapeDtypeStruct((NF, LAN), jnp.float32),
    ],
    scratch_types=[
        pltpu.VMEM((2, 3, 3 * CF), jnp.int32),
        pltpu.VMEM((2, 3, 3 * CF + 16), jnp.float32),
        pltpu.VMEM((2, 3 * CF + 16), jnp.float32),
        pltpu.VMEM((2, 3 * CF + 16), jnp.float32),
        pltpu.VMEM((2, 9 * CF, LAN), jnp.float32),
        pltpu.VMEM((2, CF, LAN), jnp.float32),
        pltpu.VMEM((2, CF, LAN), jnp.float32),
        pltpu.SemaphoreType.DMA,
        pltpu.SemaphoreType.DMA,
        pltpu.SemaphoreType.DMA,
        pltpu.SemaphoreType.DMA,
        pltpu.SemaphoreType.DMA,
        pltpu.SemaphoreType.DMA,
    ],
)
def _stage_a(xt, gcols, gvals, ewf, nsf, oew, ons,
             idxb, gvb, ewb, nsb, rowsb, oeb, onb,
             si0, si1, sg0, sg1, so0, so1):
    wid = _wid()
    f0 = wid * FPT
    sis = (si0, si1)
    sgs = (sg0, sg1)
    sos = (so0, so1)

    def start_idx(ch, b):
        fb = f0 + ch * CF
        for d in range(3):
            pltpu.async_copy(gcols.at[pl.ds(3 * d * NF + 3 * fb, 3 * CF)],
                             idxb.at[b, d], sis[b])

    def wait_idx(b):
        for d in range(3):
            pltpu.make_async_copy(gcols.at[pl.ds(0, 3 * CF)],
                                  idxb.at[b, d], sis[b]).wait()

    def start_gather(ch, b):
        fb = f0 + ch * CF
        for d in range(3):
            pltpu.async_copy(xt.at[idxb.at[b, d]],
                             rowsb.at[b, pl.ds(d * 3 * CF, 3 * CF)], sgs[b])
            pltpu.async_copy(gvals.at[pl.ds(3 * d * NF + 3 * fb, 3 * CF)],
                             gvb.at[b, d, pl.ds(0, 3 * CF)], sgs[b])
        pltpu.async_copy(ewf.at[pl.ds(3 * fb, 3 * CF)],
                         ewb.at[b, pl.ds(0, 3 * CF)], sgs[b])
        pltpu.async_copy(nsf.at[pl.ds(3 * fb, 3 * CF)],
                         nsb.at[b, pl.ds(0, 3 * CF)], sgs[b])

    def wait_gather(b):
        for d in range(3):
            pltpu.make_async_copy(xt.at[pl.ds(0, 3 * CF)],
                                  rowsb.at[b, pl.ds(d * 3 * CF, 3 * CF)],
                                  sgs[b]).wait()
            pltpu.make_async_copy(gvals.at[pl.ds(0, 3 * CF)],
                                  gvb.at[b, d, pl.ds(0, 3 * CF)], sgs[b]).wait()
        pltpu.make_async_copy(ewf.at[pl.ds(0, 3 * CF)],
                              ewb.at[b, pl.ds(0, 3 * CF)], sgs[b]).wait()
        pltpu.make_async_copy(nsf.at[pl.ds(0, 3 * CF)],
                              nsb.at[b, pl.ds(0, 3 * CF)], sgs[b]).wait()

    def start_out(ch, b):
        fb = f0 + ch * CF
        pltpu.async_copy(oeb.at[b], oew.at[pl.ds(fb, CF)], sos[b])
        pltpu.async_copy(onb.at[b], ons.at[pl.ds(fb, CF)], sos[b])

    def wait_out(b):
        pltpu.make_async_copy(oeb.at[b], oew.at[pl.ds(0, CF)], sos[b]).wait()
        pltpu.make_async_copy(onb.at[b], ons.at[pl.ds(0, CF)], sos[b]).wait()

    def compute(b):
        def face(fl, inner):
            acc_e = [jnp.zeros((16,), jnp.float32) for _ in range(NSEG)]
            acc_n = [jnp.zeros((16,), jnp.float32) for _ in range(NSEG)]
            ew_vec = ewb[b, pl.ds(3 * fl, 16)]
            ns_vec = nsb[b, pl.ds(3 * fl, 16)]
            gv_vecs = [gvb[b, d, pl.ds(3 * fl, 16)] for d in range(3)]
            for d in range(3):
                we_d = ew_vec[d]
                wn_d = ns_vec[d]
                for j in range(3):
                    g = gv_vecs[d][j]
                    we = we_d * g
                    wn = wn_d * g
                    r = d * 3 * CF + 3 * fl + j
                    for s in range(NSEG):
                        seg = rowsb[b, r, pl.ds(16 * s, 16)]
                        acc_e[s] = acc_e[s] + we * seg
                        acc_n[s] = acc_n[s] + wn * seg
            for s in range(NSEG):
                oeb[b, fl, pl.ds(16 * s, 16)] = acc_e[s]
                onb[b, fl, pl.ds(16 * s, 16)] = acc_n[s]
            return inner

        lax.fori_loop(0, CF, face, 0)

    start_idx(0, 0)
    wait_idx(0)
    start_gather(0, 0)
    start_idx(1, 1)

    def body2(g, carry):
        for b in range(2):
            ch = g + b
            wait_gather(b)

            @pl.when(ch + 1 < A_CHUNKS)
            def _():
                wait_idx(1 - b)
                start_gather(ch + 1, 1 - b)

            @pl.when(ch + 2 < A_CHUNKS)
            def _():
                start_idx(ch + 2, b)

            @pl.when(ch >= 2)
            def _():
                wait_out(b)

            compute(b)
            start_out(ch, b)
        return carry

    lax.fori_loop(0, A_CHUNKS // 2, lambda i, c: body2(2 * i, c), 0)
    wait_out(0)
    wait_out(1)


@functools.partial(
    pl.kernel,
    mesh=_mesh,
    out_type=[
        jax.ShapeDtypeStruct((NV_PAD, LAN), jnp.float32),
        jax.ShapeDtypeStruct((NV_PAD, LAN), jnp.float32),
        jax.ShapeDtypeStruct((NV_PAD, LAN), jnp.float32),
    ],
    scratch_types=[
        pltpu.VMEM((2, 6 * CV), jnp.int32),
        pltpu.VMEM((2, 6 * CV + 16), jnp.float32),
        pltpu.VMEM((2, 7 * CV), jnp.int32),
        pltpu.VMEM((2, 7 * CV + 16), jnp.float32),
        pltpu.VMEM((2, 6 * CV, LAN), jnp.float32),
        pltpu.VMEM((2, 6 * CV, LAN), jnp.float32),
        pltpu.VMEM((2, 7 * CV, LAN), jnp.float32),
        pltpu.VMEM((2, CV, LAN), jnp.float32),
        pltpu.VMEM((2, CV, LAN), jnp.float32),
        pltpu.VMEM((2, CV, LAN), jnp.float32),
        pltpu.SemaphoreType.DMA,
        pltpu.SemaphoreType.DMA,
        pltpu.SemaphoreType.DMA,
        pltpu.SemaphoreType.DMA,
        pltpu.SemaphoreType.DMA,
        pltpu.SemaphoreType.DMA,
    ],
)
def _stage_b(gfe, gfn, xt, fcols, fvals, lcols, lvals,
             ove, ovn, olap,
             fib, fvb, lib, lvb, reb, rnb, rlb, oeb, onb, olb,
             si0, si1, sg0, sg1, so0, so1):
    wid = _wid()
    v0 = wid * VPT
    sis = (si0, si1)
    sgs = (sg0, sg1)
    sos = (so0, so1)

    def start_idx(ch, b):
        vb = v0 + ch * CV
        pltpu.async_copy(fcols.at[pl.ds(6 * vb, 6 * CV)], fib.at[b], sis[b])
        pltpu.async_copy(lcols.at[pl.ds(7 * vb, 7 * CV)], lib.at[b], sis[b])

    def wait_idx(b):
        pltpu.make_async_copy(fcols.at[pl.ds(0, 6 * CV)], fib.at[b], sis[b]).wait()
        pltpu.make_async_copy(lcols.at[pl.ds(0, 7 * CV)], lib.at[b], sis[b]).wait()

    def start_gather(ch, b):
        vb = v0 + ch * CV
        pltpu.async_copy(gfe.at[fib.at[b]], reb.at[b], sgs[b])
        pltpu.async_copy(gfn.at[fib.at[b]], rnb.at[b], sgs[b])
        pltpu.async_copy(xt.at[lib.at[b]], rlb.at[b], sgs[b])
        pltpu.async_copy(fvals.at[pl.ds(6 * vb, 6 * CV)],
                         fvb.at[b, pl.ds(0, 6 * CV)], sgs[b])
        pltpu.async_copy(lvals.at[pl.ds(7 * vb, 7 * CV)],
                         lvb.at[b, pl.ds(0, 7 * CV)], sgs[b])

    def wait_gather(b):
        pltpu.make_async_copy(gfe.at[pl.ds(0, 6 * CV)], reb.at[b], sgs[b]).wait()
        pltpu.make_async_copy(gfn.at[pl.ds(0, 6 * CV)], rnb.at[b], sgs[b]).wait()
        pltpu.make_async_copy(xt.at[pl.ds(0, 7 * CV)], rlb.at[b], sgs[b]).wait()
        pltpu.make_async_copy(fvals.at[pl.ds(0, 6 * CV)],
                              fvb.at[b, pl.ds(0, 6 * CV)], sgs[b]).wait()
        pltpu.make_async_copy(lvals.at[pl.ds(0, 7 * CV)],
                              lvb.at[b, pl.ds(0, 7 * CV)], sgs[b]).wait()

    def start_out(ch, b):
        vb = v0 + ch * CV
        pltpu.async_copy(oeb.at[b], ove.at[pl.ds(vb, CV)], sos[b])
        pltpu.async_copy(onb.at[b], ovn.at[pl.ds(vb, CV)], sos[b])
        pltpu.async_copy(olb.at[b], olap.at[pl.ds(vb, CV)], sos[b])

    def wait_out(b):
        pltpu.make_async_copy(oeb.at[b], ove.at[pl.ds(0, CV)], sos[b]).wait()
        pltpu.make_async_copy(onb.at[b], ovn.at[pl.ds(0, CV)], sos[b]).wait()
        pltpu.make_async_copy(olb.at[b], olap.at[pl.ds(0, CV)], sos[b]).wait()

    def compute(b):
        def vert(vl, inner):
            acc_e = [jnp.zeros((16,), jnp.float32) for _ in range(NSEG)]
            acc_n = [jnp.zeros((16,), jnp.float32) for _ in range(NSEG)]
            acc_l = [jnp.zeros((16,), jnp.float32) for _ in range(NSEG)]
            fv_vec = fvb[b, pl.ds(6 * vl, 16)]
            lv_vec = lvb[b, pl.ds(7 * vl, 16)]
            for j in range(6):
                r = 6 * vl + j
                w = fv_vec[j]
                for s in range(NSEG):
                    sl = pl.ds(16 * s, 16)
                    acc_e[s] = acc_e[s] + w * reb[b, r, sl]
                    acc_n[s] = acc_n[s] + w * rnb[b, r, sl]
            for j in range(7):
                r = 7 * vl + j
                w = lv_vec[j]
                for s in range(NSEG):
                    acc_l[s] = acc_l[s] + w * rlb[b, r, pl.ds(16 * s, 16)]
            for s in range(NSEG):
                sl = pl.ds(16 * s, 16)
                oeb[b, vl, sl] = acc_e[s]
                onb[b, vl, sl] = acc_n[s]
                olb[b, vl, sl] = acc_l[s]
            return inner

        lax.fori_loop(0, CV, vert, 0)

    start_idx(0, 0)
    wait_idx(0)
    start_gather(0, 0)
    start_idx(1, 1)

    def body2(g, carry):
        for b in range(2):
            ch = g + b
            wait_gather(b)

            @pl.when(ch + 1 < B_CHUNKS)
            def _():
                wait_idx(1 - b)
                start_gather(ch + 1, 1 - b)

            @pl.when(ch + 2 < B_CHUNKS)
            def _():
                start_idx(ch + 2, b)

            @pl.when(ch >= 2)
            def _():
                wait_out(b)

            compute(b)
            start_out(ch, b)
        return carry

    lax.fori_loop(0, B_CHUNKS // 2, lambda i, c: body2(2 * i, c), 0)
    wait_out(0)
    wait_out(1)


TR = 512
GRID_D = (NV + TR - 1) // TR


def _stage_d_body(id_r, ew_r, ns_r, lp_r, bd_r, out_r):
    acc = jnp.dot(id_r[...], bd_r[0], preferred_element_type=jnp.float32)
    acc = acc + jnp.dot(ew_r[...], bd_r[1], preferred_element_type=jnp.float32)
    acc = acc + jnp.dot(ns_r[...], bd_r[2], preferred_element_type=jnp.float32)
    acc = acc + jnp.dot(lp_r[...], bd_r[3], preferred_element_type=jnp.float32)
    out_r[...] = acc


def _stage_d(x_t, gve, gvn, lap, bd):
    row_spec = pl.BlockSpec((TR, LAN), lambda i: (i, 0))
    return pl.pallas_call(
        _stage_d_body,
        grid=(GRID_D,),
        in_specs=[row_spec, row_spec, row_spec, row_spec,
                  pl.BlockSpec((4, LAN, B * COUT), lambda i: (0, 0, 0))],
        out_specs=pl.BlockSpec((TR, B * COUT), lambda i: (i, 0)),
        out_shape=jax.ShapeDtypeStruct((NV, B * COUT), jnp.float32),
    )(x_t, gve, gvn, lap, bd)


def kernel(x, coeffs, G_rows, G_cols, G_vals, EW, NS,
           L_rows, L_cols, L_vals, F_rows, F_cols, F_vals):
    x_t = x.transpose(2, 0, 1).reshape(NV, LAN)
    ewf = EW.reshape(-1)
    nsf = NS.reshape(-1)
    pad_f = NV_PAD * 6 - NV * 6
    pad_l = NV_PAD * 7 - NV * 7
    fcols = jnp.concatenate([F_cols, jnp.zeros((pad_f,), jnp.int32)])
    fvals = jnp.concatenate([F_vals, jnp.zeros((pad_f,), jnp.float32)])
    lcols = jnp.concatenate([L_cols, jnp.zeros((pad_l,), jnp.int32)])
    lvals = jnp.concatenate([L_vals, jnp.zeros((pad_l,), jnp.float32)])

    gfe, gfn = _stage_a(x_t, G_cols, G_vals, ewf, nsf)
    gve, gvn, lap = _stage_b(gfe, gfn, x_t, fcols, fvals, lcols, lvals)

    wk = coeffs.transpose(2, 1, 0)
    eye = jnp.eye(B, dtype=jnp.float32)
    bd = jax.vmap(lambda w: jnp.kron(eye, w))(wk)

    out_t = _stage_d(x_t, gve, gvn, lap, bd)
    return out_t.reshape(NV, B, COUT).transpose(1, 2, 0)


# speedup vs baseline: 165.1017x; 1.0010x over previous
import functools

import jax
import jax.numpy as jnp
from jax import lax
from jax.experimental import pallas as pl
from jax.experimental.pallas import tpu as pltpu
from jax.experimental.pallas import tpu_sc as plsc

NV = 40962
NF = 81920
B = 4
CIN = 32
COUT = 32
LAN = B * CIN
NSEG = LAN // 16

NW = 32
CF = 32
FPT = NF // NW
A_CHUNKS = FPT // CF

CV = 16
VPT = 1312
NV_PAD = NW * VPT
B_CHUNKS = VPT // CV

_mesh = plsc.VectorSubcoreMesh(core_axis_name="c", subcore_axis_name="s")


def _wid():
    return lax.axis_index("s") * 2 + lax.axis_index("c")


@functools.partial(
    pl.kernel,
    mesh=_mesh,
    out_type=[
        jax.ShapeDtypeStruct((NF, 2 * LAN), jnp.float32),
    ],
    scratch_types=[
        pltpu.VMEM((2, 3, 3 * CF), jnp.int32),
        pltpu.VMEM((2, 3, 3 * CF + 16), jnp.float32),
        pltpu.VMEM((2, 3 * CF + 16), jnp.float32),
        pltpu.VMEM((2, 3 * CF + 16), jnp.float32),
        pltpu.VMEM((2, 9 * CF, LAN), jnp.float32),
        pltpu.VMEM((2, CF, 2 * LAN), jnp.float32),
        pltpu.SemaphoreType.DMA,
        pltpu.SemaphoreType.DMA,
        pltpu.SemaphoreType.DMA,
        pltpu.SemaphoreType.DMA,
        pltpu.SemaphoreType.DMA,
        pltpu.SemaphoreType.DMA,
    ],
)
def _stage_a(xt, gcols, gvals, ewf, nsf, gf2,
             idxb, gvb, ewb, nsb, rowsb, o2b,
             si0, si1, sg0, sg1, so0, so1):
    wid = _wid()
    f0 = wid * FPT
    sis = (si0, si1)
    sgs = (sg0, sg1)
    sos = (so0, so1)

    def start_idx(ch, b):
        fb = f0 + ch * CF
        for d in range(3):
            pltpu.async_copy(gcols.at[pl.ds(3 * d * NF + 3 * fb, 3 * CF)],
                             idxb.at[b, d], sis[b])

    def wait_idx(b):
        for d in range(3):
            pltpu.make_async_copy(gcols.at[pl.ds(0, 3 * CF)],
                                  idxb.at[b, d], sis[b]).wait()

    def start_gather(ch, b):
        fb = f0 + ch * CF
        for d in range(3):
            pltpu.async_copy(xt.at[idxb.at[b, d]],
                             rowsb.at[b, pl.ds(d * 3 * CF, 3 * CF)], sgs[b])
            pltpu.async_copy(gvals.at[pl.ds(3 * d * NF + 3 * fb, 3 * CF)],
                             gvb.at[b, d, pl.ds(0, 3 * CF)], sgs[b])
        pltpu.async_copy(ewf.at[pl.ds(3 * fb, 3 * CF)],
                         ewb.at[b, pl.ds(0, 3 * CF)], sgs[b])
        pltpu.async_copy(nsf.at[pl.ds(3 * fb, 3 * CF)],
                         nsb.at[b, pl.ds(0, 3 * CF)], sgs[b])

    def wait_gather(b):
        for d in range(3):
            pltpu.make_async_copy(xt.at[pl.ds(0, 3 * CF)],
                                  rowsb.at[b, pl.ds(d * 3 * CF, 3 * CF)],
                                  sgs[b]).wait()
            pltpu.make_async_copy(gvals.at[pl.ds(0, 3 * CF)],
                                  gvb.at[b, d, pl.ds(0, 3 * CF)], sgs[b]).wait()
        pltpu.make_async_copy(ewf.at[pl.ds(0, 3 * CF)],
                              ewb.at[b, pl.ds(0, 3 * CF)], sgs[b]).wait()
        pltpu.make_async_copy(nsf.at[pl.ds(0, 3 * CF)],
                              nsb.at[b, pl.ds(0, 3 * CF)], sgs[b]).wait()

    def start_out(ch, b):
        fb = f0 + ch * CF
        pltpu.async_copy(o2b.at[b], gf2.at[pl.ds(fb, CF)], sos[b])

    def wait_out(b):
        pltpu.make_async_copy(o2b.at[b], gf2.at[pl.ds(0, CF)], sos[b]).wait()

    def compute(b):
        def face(fl, inner):
            acc_e = [jnp.zeros((16,), jnp.float32) for _ in range(NSEG)]
            acc_n = [jnp.zeros((16,), jnp.float32) for _ in range(NSEG)]
            ew_vec = ewb[b, pl.ds(3 * fl, 16)]
            ns_vec = nsb[b, pl.ds(3 * fl, 16)]
            gv_vecs = [gvb[b, d, pl.ds(3 * fl, 16)] for d in range(3)]
            for d in range(3):
                we_d = ew_vec[d]
                wn_d = ns_vec[d]
                for j in range(3):
                    g = gv_vecs[d][j]
                    we = we_d * g
                    wn = wn_d * g
                    r = d * 3 * CF + 3 * fl + j
                    for s in range(NSEG):
                        seg = rowsb[b, r, pl.ds(16 * s, 16)]
                        acc_e[s] = acc_e[s] + we * seg
                        acc_n[s] = acc_n[s] + wn * seg
            for s in range(NSEG):
                o2b[b, fl, pl.ds(16 * s, 16)] = acc_e[s]
                o2b[b, fl, pl.ds(LAN + 16 * s, 16)] = acc_n[s]
            return inner

        lax.fori_loop(0, CF, face, 0)

    start_idx(0, 0)
    wait_idx(0)
    start_gather(0, 0)
    start_idx(1, 1)

    def body2(g, carry):
        for b in range(2):
            ch = g + b
            wait_gather(b)

            @pl.when(ch + 1 < A_CHUNKS)
            def _():
                wait_idx(1 - b)
                start_gather(ch + 1, 1 - b)

            @pl.when(ch + 2 < A_CHUNKS)
            def _():
                start_idx(ch + 2, b)

            @pl.when(ch >= 2)
            def _():
                wait_out(b)

            compute(b)
            start_out(ch, b)
        return carry

    lax.fori_loop(0, A_CHUNKS // 2, lambda i, c: body2(2 * i, c), 0)
    wait_out(0)
    wait_out(1)


@functools.partial(
    pl.kernel,
    mesh=_mesh,
    out_type=[
        jax.ShapeDtypeStruct((NV_PAD, 3 * LAN), jnp.float32),
    ],
    scratch_types=[
        pltpu.VMEM((2, 6 * CV), jnp.int32),
        pltpu.VMEM((2, 6 * CV + 16), jnp.float32),
        pltpu.VMEM((2, 7 * CV), jnp.int32),
        pltpu.VMEM((2, 7 * CV + 16), jnp.float32),
        pltpu.VMEM((2, 6 * CV, 2 * LAN), jnp.float32),
        pltpu.VMEM((2, 7 * CV, LAN), jnp.float32),
        pltpu.VMEM((2, CV, 3 * LAN), jnp.float32),
        pltpu.SemaphoreType.DMA,
        pltpu.SemaphoreType.DMA,
        pltpu.SemaphoreType.DMA,
        pltpu.SemaphoreType.DMA,
        pltpu.SemaphoreType.DMA,
        pltpu.SemaphoreType.DMA,
    ],
)
def _stage_b(gf2, xt, fcols, fvals, lcols, lvals,
             gout,
             fib, fvb, lib, lvb, r2b, rlb, o3b,
             si0, si1, sg0, sg1, so0, so1):
    wid = _wid()
    v0 = wid * VPT
    sis = (si0, si1)
    sgs = (sg0, sg1)
    sos = (so0, so1)

    def start_idx(ch, b):
        vb = v0 + ch * CV
        pltpu.async_copy(fcols.at[pl.ds(6 * vb, 6 * CV)], fib.at[b], sis[b])
        pltpu.async_copy(lcols.at[pl.ds(7 * vb, 7 * CV)], lib.at[b], sis[b])

    def wait_idx(b):
        pltpu.make_async_copy(fcols.at[pl.ds(0, 6 * CV)], fib.at[b], sis[b]).wait()
        pltpu.make_async_copy(lcols.at[pl.ds(0, 7 * CV)], lib.at[b], sis[b]).wait()

    def start_gather(ch, b):
        vb = v0 + ch * CV
        pltpu.async_copy(gf2.at[fib.at[b]], r2b.at[b], sgs[b])
        pltpu.async_copy(xt.at[lib.at[b]], rlb.at[b], sgs[b])
        pltpu.async_copy(fvals.at[pl.ds(6 * vb, 6 * CV)],
                         fvb.at[b, pl.ds(0, 6 * CV)], sgs[b])
        pltpu.async_copy(lvals.at[pl.ds(7 * vb, 7 * CV)],
                         lvb.at[b, pl.ds(0, 7 * CV)], sgs[b])

    def wait_gather(b):
        pltpu.make_async_copy(gf2.at[pl.ds(0, 6 * CV)], r2b.at[b], sgs[b]).wait()
        pltpu.make_async_copy(xt.at[pl.ds(0, 7 * CV)], rlb.at[b], sgs[b]).wait()
        pltpu.make_async_copy(fvals.at[pl.ds(0, 6 * CV)],
                              fvb.at[b, pl.ds(0, 6 * CV)], sgs[b]).wait()
        pltpu.make_async_copy(lvals.at[pl.ds(0, 7 * CV)],
                              lvb.at[b, pl.ds(0, 7 * CV)], sgs[b]).wait()

    def start_out(ch, b):
        vb = v0 + ch * CV
        pltpu.async_copy(o3b.at[b], gout.at[pl.ds(vb, CV)], sos[b])

    def wait_out(b):
        pltpu.make_async_copy(o3b.at[b], gout.at[pl.ds(0, CV)], sos[b]).wait()

    def compute(b):
        def vert(vl, inner):
            acc_e = [jnp.zeros((16,), jnp.float32) for _ in range(NSEG)]
            acc_n = [jnp.zeros((16,), jnp.float32) for _ in range(NSEG)]
            acc_l = [jnp.zeros((16,), jnp.float32) for _ in range(NSEG)]
            fv_vec = fvb[b, pl.ds(6 * vl, 16)]
            lv_vec = lvb[b, pl.ds(7 * vl, 16)]
            for j in range(6):
                r = 6 * vl + j
                w = fv_vec[j]
                for s in range(NSEG):
                    acc_e[s] = acc_e[s] + w * r2b[b, r, pl.ds(16 * s, 16)]
                    acc_n[s] = acc_n[s] + w * r2b[b, r, pl.ds(LAN + 16 * s, 16)]
            for j in range(7):
                r = 7 * vl + j
                w = lv_vec[j]
                for s in range(NSEG):
                    acc_l[s] = acc_l[s] + w * rlb[b, r, pl.ds(16 * s, 16)]
            for s in range(NSEG):
                o3b[b, vl, pl.ds(16 * s, 16)] = acc_e[s]
                o3b[b, vl, pl.ds(LAN + 16 * s, 16)] = acc_n[s]
                o3b[b, vl, pl.ds(2 * LAN + 16 * s, 16)] = acc_l[s]
            return inner

        lax.fori_loop(0, CV, vert, 0)

    start_idx(0, 0)
    wait_idx(0)
    start_gather(0, 0)
    start_idx(1, 1)

    def body2(g, carry):
        for b in range(2):
            ch = g + b
            wait_gather(b)

            @pl.when(ch + 1 < B_CHUNKS)
            def _():
                wait_idx(1 - b)
                start_gather(ch + 1, 1 - b)

            @pl.when(ch + 2 < B_CHUNKS)
            def _():
                start_idx(ch + 2, b)

            @pl.when(ch >= 2)
            def _():
                wait_out(b)

            compute(b)
            start_out(ch, b)
        return carry

    lax.fori_loop(0, B_CHUNKS // 2, lambda i, c: body2(2 * i, c), 0)
    wait_out(0)
    wait_out(1)


TR = 512
GRID_D = (NV + TR - 1) // TR


def _stage_d_body(id_r, gv_r, bd_r, out_r):
    gv = gv_r[...]
    acc = jnp.dot(id_r[...], bd_r[0], preferred_element_type=jnp.float32)
    acc = acc + jnp.dot(gv[:, :LAN], bd_r[1], preferred_element_type=jnp.float32)
    acc = acc + jnp.dot(gv[:, LAN:2 * LAN], bd_r[2],
                        preferred_element_type=jnp.float32)
    acc = acc + jnp.dot(gv[:, 2 * LAN:], bd_r[3],
                        preferred_element_type=jnp.float32)
    out_r[...] = acc


def _stage_d(x_t, gout, bd):
    return pl.pallas_call(
        _stage_d_body,
        grid=(GRID_D,),
        in_specs=[pl.BlockSpec((TR, LAN), lambda i: (i, 0)),
                  pl.BlockSpec((TR, 3 * LAN), lambda i: (i, 0)),
                  pl.BlockSpec((4, LAN, B * COUT), lambda i: (0, 0, 0))],
        out_specs=pl.BlockSpec((TR, B * COUT), lambda i: (i, 0)),
        out_shape=jax.ShapeDtypeStruct((NV, B * COUT), jnp.float32),
    )(x_t, gout, bd)


def kernel(x, coeffs, G_rows, G_cols, G_vals, EW, NS,
           L_rows, L_cols, L_vals, F_rows, F_cols, F_vals):
    x_t = x.transpose(2, 0, 1).reshape(NV, LAN)
    ewf = EW.reshape(-1)
    nsf = NS.reshape(-1)
    pad_f = NV_PAD * 6 - NV * 6
    pad_l = NV_PAD * 7 - NV * 7
    fcols = jnp.concatenate([F_cols, jnp.zeros((pad_f,), jnp.int32)])
    fvals = jnp.concatenate([F_vals, jnp.zeros((pad_f,), jnp.float32)])
    lcols = jnp.concatenate([L_cols, jnp.zeros((pad_l,), jnp.int32)])
    lvals = jnp.concatenate([L_vals, jnp.zeros((pad_l,), jnp.float32)])

    gf2 = _stage_a(x_t, G_cols, G_vals, ewf, nsf)
    if isinstance(gf2, (list, tuple)):
        gf2 = gf2[0]
    gout = _stage_b(gf2, x_t, fcols, fvals, lcols, lvals)
    if isinstance(gout, (list, tuple)):
        gout = gout[0]

    wk = coeffs.transpose(2, 1, 0)
    eye = jnp.eye(B, dtype=jnp.float32)
    bd = jax.vmap(lambda w: jnp.kron(eye, w))(wk)

    out_t = _stage_d(x_t, gout, bd)
    return out_t.reshape(NV, B, COUT).transpose(1, 2, 0)


# speedup vs baseline: 292.1515x; 1.7695x over previous
import functools

import jax
import jax.numpy as jnp
from jax import lax
from jax.experimental import pallas as pl
from jax.experimental.pallas import tpu as pltpu
from jax.experimental.pallas import tpu_sc as plsc

NV = 40962
NF = 81920
B = 4
CIN = 32
COUT = 32
LAN = B * CIN
NSEG = LAN // 16

NW = 32
CF = 32
FPT = NF // NW
A_CHUNKS = FPT // CF

CV = 16
VPT = 1312
NV_PAD = NW * VPT
B_CHUNKS = VPT // CV

_mesh = plsc.VectorSubcoreMesh(core_axis_name="c", subcore_axis_name="s")


def _wid():
    return lax.axis_index("s") * 2 + lax.axis_index("c")


@functools.partial(
    pl.kernel,
    mesh=_mesh,
    out_type=[
        jax.ShapeDtypeStruct((NF, 2 * LAN), jnp.float32),
    ],
    scratch_types=[
        pltpu.VMEM((2, 3, 3 * CF), jnp.int32),
        pltpu.VMEM((2, 3, 3 * CF + 16), jnp.float32),
        pltpu.VMEM((2, 3 * CF + 16), jnp.float32),
        pltpu.VMEM((2, 3 * CF + 16), jnp.float32),
        pltpu.VMEM((2, 9 * CF, LAN), jnp.float32),
        pltpu.VMEM((2, CF, 2 * LAN), jnp.float32),
        pltpu.SemaphoreType.DMA,
        pltpu.SemaphoreType.DMA,
        pltpu.SemaphoreType.DMA,
        pltpu.SemaphoreType.DMA,
        pltpu.SemaphoreType.DMA,
        pltpu.SemaphoreType.DMA,
    ],
)
def _stage_a(xt, gcols, gvals, ewf, nsf, gf2,
             idxb, gvb, ewb, nsb, rowsb, o2b,
             si0, si1, sg0, sg1, so0, so1):
    wid = _wid()
    f0 = wid * FPT
    sis = (si0, si1)
    sgs = (sg0, sg1)
    sos = (so0, so1)

    def start_idx(ch, b):
        fb = f0 + ch * CF
        for d in range(3):
            pltpu.async_copy(gcols.at[pl.ds(3 * d * NF + 3 * fb, 3 * CF)],
                             idxb.at[b, d], sis[b])

    def wait_idx(b):
        for d in range(3):
            pltpu.make_async_copy(gcols.at[pl.ds(0, 3 * CF)],
                                  idxb.at[b, d], sis[b]).wait()

    def start_gather(ch, b):
        fb = f0 + ch * CF
        for d in range(3):
            pltpu.async_copy(xt.at[idxb.at[b, d]],
                             rowsb.at[b, pl.ds(d * 3 * CF, 3 * CF)], sgs[b])
            pltpu.async_copy(gvals.at[pl.ds(3 * d * NF + 3 * fb, 3 * CF)],
                             gvb.at[b, d, pl.ds(0, 3 * CF)], sgs[b])
        pltpu.async_copy(ewf.at[pl.ds(3 * fb, 3 * CF)],
                         ewb.at[b, pl.ds(0, 3 * CF)], sgs[b])
        pltpu.async_copy(nsf.at[pl.ds(3 * fb, 3 * CF)],
                         nsb.at[b, pl.ds(0, 3 * CF)], sgs[b])

    def wait_gather(b):
        for d in range(3):
            pltpu.make_async_copy(xt.at[pl.ds(0, 3 * CF)],
                                  rowsb.at[b, pl.ds(d * 3 * CF, 3 * CF)],
                                  sgs[b]).wait()
            pltpu.make_async_copy(gvals.at[pl.ds(0, 3 * CF)],
                                  gvb.at[b, d, pl.ds(0, 3 * CF)], sgs[b]).wait()
        pltpu.make_async_copy(ewf.at[pl.ds(0, 3 * CF)],
                              ewb.at[b, pl.ds(0, 3 * CF)], sgs[b]).wait()
        pltpu.make_async_copy(nsf.at[pl.ds(0, 3 * CF)],
                              nsb.at[b, pl.ds(0, 3 * CF)], sgs[b]).wait()

    def start_out(ch, b):
        fb = f0 + ch * CF
        pltpu.async_copy(o2b.at[b], gf2.at[pl.ds(fb, CF)], sos[b])

    def wait_out(b):
        pltpu.make_async_copy(o2b.at[b], gf2.at[pl.ds(0, CF)], sos[b]).wait()

    def compute(b):
        def face(fl, inner):
            acc_e = [jnp.zeros((16,), jnp.float32) for _ in range(NSEG)]
            acc_n = [jnp.zeros((16,), jnp.float32) for _ in range(NSEG)]
            ew_vec = ewb[b, pl.ds(3 * fl, 16)]
            ns_vec = nsb[b, pl.ds(3 * fl, 16)]
            gv_vecs = [gvb[b, d, pl.ds(3 * fl, 16)] for d in range(3)]
            for d in range(3):
                we_d = ew_vec[d]
                wn_d = ns_vec[d]
                for j in range(3):
                    g = gv_vecs[d][j]
                    we = we_d * g
                    wn = wn_d * g
                    r = d * 3 * CF + 3 * fl + j
                    for s in range(NSEG):
                        seg = rowsb[b, r, pl.ds(16 * s, 16)]
                        acc_e[s] = acc_e[s] + we * seg
                        acc_n[s] = acc_n[s] + wn * seg
            for s in range(NSEG):
                o2b[b, fl, pl.ds(16 * s, 16)] = acc_e[s]
                o2b[b, fl, pl.ds(LAN + 16 * s, 16)] = acc_n[s]
            return inner

        lax.fori_loop(0, CF, face, 0)

    start_idx(0, 0)
    wait_idx(0)
    start_gather(0, 0)
    start_idx(1, 1)

    def body2(g, carry):
        for b in range(2):
            ch = g + b
            wait_gather(b)

            @pl.when(ch + 1 < A_CHUNKS)
            def _():
                wait_idx(1 - b)
                start_gather(ch + 1, 1 - b)

            @pl.when(ch + 2 < A_CHUNKS)
            def _():
                start_idx(ch + 2, b)

            @pl.when(ch >= 2)
            def _():
                wait_out(b)

            compute(b)
            start_out(ch, b)
        return carry

    lax.fori_loop(0, A_CHUNKS // 2, lambda i, c: body2(2 * i, c), 0)
    wait_out(0)
    wait_out(1)


@functools.partial(
    pl.kernel,
    mesh=_mesh,
    out_type=[
        jax.ShapeDtypeStruct((NV_PAD, 3 * LAN), jnp.float32),
    ],
    scratch_types=[
        pltpu.VMEM((2, 6 * CV), jnp.int32),
        pltpu.VMEM((2, 6 * CV + 16), jnp.float32),
        pltpu.VMEM((2, 7 * CV), jnp.int32),
        pltpu.VMEM((2, 7 * CV + 16), jnp.float32),
        pltpu.VMEM((2, 6 * CV, 2 * LAN), jnp.float32),
        pltpu.VMEM((2, 7 * CV, LAN), jnp.float32),
        pltpu.VMEM((2, CV, 3 * LAN), jnp.float32),
        pltpu.SemaphoreType.DMA,
        pltpu.SemaphoreType.DMA,
        pltpu.SemaphoreType.DMA,
        pltpu.SemaphoreType.DMA,
        pltpu.SemaphoreType.DMA,
        pltpu.SemaphoreType.DMA,
    ],
)
def _stage_b(gf2, xt, fcols, fvals, lcols, lvals,
             gout,
             fib, fvb, lib, lvb, r2b, rlb, o3b,
             si0, si1, sg0, sg1, so0, so1):
    wid = _wid()
    v0 = wid * VPT
    sis = (si0, si1)
    sgs = (sg0, sg1)
    sos = (so0, so1)

    def start_idx(ch, b):
        vb = v0 + ch * CV
        pltpu.async_copy(fcols.at[pl.ds(6 * vb, 6 * CV)], fib.at[b], sis[b])
        pltpu.async_copy(lcols.at[pl.ds(7 * vb, 7 * CV)], lib.at[b], sis[b])

    def wait_idx(b):
        pltpu.make_async_copy(fcols.at[pl.ds(0, 6 * CV)], fib.at[b], sis[b]).wait()
        pltpu.make_async_copy(lcols.at[pl.ds(0, 7 * CV)], lib.at[b], sis[b]).wait()

    def start_gather(ch, b):
        vb = v0 + ch * CV
        pltpu.async_copy(gf2.at[fib.at[b]], r2b.at[b], sgs[b])
        pltpu.async_copy(xt.at[lib.at[b]], rlb.at[b], sgs[b])
        pltpu.async_copy(fvals.at[pl.ds(6 * vb, 6 * CV)],
                         fvb.at[b, pl.ds(0, 6 * CV)], sgs[b])
        pltpu.async_copy(lvals.at[pl.ds(7 * vb, 7 * CV)],
                         lvb.at[b, pl.ds(0, 7 * CV)], sgs[b])

    def wait_gather(b):
        pltpu.make_async_copy(gf2.at[pl.ds(0, 6 * CV)], r2b.at[b], sgs[b]).wait()
        pltpu.make_async_copy(xt.at[pl.ds(0, 7 * CV)], rlb.at[b], sgs[b]).wait()
        pltpu.make_async_copy(fvals.at[pl.ds(0, 6 * CV)],
                              fvb.at[b, pl.ds(0, 6 * CV)], sgs[b]).wait()
        pltpu.make_async_copy(lvals.at[pl.ds(0, 7 * CV)],
                              lvb.at[b, pl.ds(0, 7 * CV)], sgs[b]).wait()

    def start_out(ch, b):
        vb = v0 + ch * CV
        pltpu.async_copy(o3b.at[b], gout.at[pl.ds(vb, CV)], sos[b])

    def wait_out(b):
        pltpu.make_async_copy(o3b.at[b], gout.at[pl.ds(0, CV)], sos[b]).wait()

    def compute(b):
        def vert(vl, inner):
            acc_e = [jnp.zeros((16,), jnp.float32) for _ in range(NSEG)]
            acc_n = [jnp.zeros((16,), jnp.float32) for _ in range(NSEG)]
            acc_l = [jnp.zeros((16,), jnp.float32) for _ in range(NSEG)]
            fv_vec = fvb[b, pl.ds(6 * vl, 16)]
            lv_vec = lvb[b, pl.ds(7 * vl, 16)]
            for j in range(6):
                r = 6 * vl + j
                w = fv_vec[j]
                for s in range(NSEG):
                    acc_e[s] = acc_e[s] + w * r2b[b, r, pl.ds(16 * s, 16)]
                    acc_n[s] = acc_n[s] + w * r2b[b, r, pl.ds(LAN + 16 * s, 16)]
            for j in range(7):
                r = 7 * vl + j
                w = lv_vec[j]
                for s in range(NSEG):
                    acc_l[s] = acc_l[s] + w * rlb[b, r, pl.ds(16 * s, 16)]
            for s in range(NSEG):
                o3b[b, vl, pl.ds(16 * s, 16)] = acc_e[s]
                o3b[b, vl, pl.ds(LAN + 16 * s, 16)] = acc_n[s]
                o3b[b, vl, pl.ds(2 * LAN + 16 * s, 16)] = acc_l[s]
            return inner

        lax.fori_loop(0, CV, vert, 0)

    start_idx(0, 0)
    wait_idx(0)
    start_gather(0, 0)
    start_idx(1, 1)

    def body2(g, carry):
        for b in range(2):
            ch = g + b
            wait_gather(b)

            @pl.when(ch + 1 < B_CHUNKS)
            def _():
                wait_idx(1 - b)
                start_gather(ch + 1, 1 - b)

            @pl.when(ch + 2 < B_CHUNKS)
            def _():
                start_idx(ch + 2, b)

            @pl.when(ch >= 2)
            def _():
                wait_out(b)

            compute(b)
            start_out(ch, b)
        return carry

    lax.fori_loop(0, B_CHUNKS // 2, lambda i, c: body2(2 * i, c), 0)
    wait_out(0)
    wait_out(1)


TR = 512
GRID_D = (NV + TR - 1) // TR


def _stage_d_body(id_r, gv_r, bd_r, out_r):
    gv = gv_r[...]
    acc = jnp.dot(id_r[...], bd_r[0], preferred_element_type=jnp.float32)
    acc = acc + jnp.dot(gv[:, :LAN], bd_r[1], preferred_element_type=jnp.float32)
    acc = acc + jnp.dot(gv[:, LAN:2 * LAN], bd_r[2],
                        preferred_element_type=jnp.float32)
    acc = acc + jnp.dot(gv[:, 2 * LAN:], bd_r[3],
                        preferred_element_type=jnp.float32)
    out_r[...] = acc


def _stage_d(x_t, gout, bd):
    return pl.pallas_call(
        _stage_d_body,
        grid=(GRID_D,),
        in_specs=[pl.BlockSpec((TR, LAN), lambda i: (i, 0)),
                  pl.BlockSpec((TR, 3 * LAN), lambda i: (i, 0)),
                  pl.BlockSpec((4, LAN, B * COUT), lambda i: (0, 0, 0))],
        out_specs=pl.BlockSpec((TR, B * COUT), lambda i: (i, 0)),
        out_shape=jax.ShapeDtypeStruct((NV, B * COUT), jnp.float32),
    )(x_t, gout, bd)


def kernel(x, coeffs, G_rows, G_cols, G_vals, EW, NS,
           L_rows, L_cols, L_vals, F_rows, F_cols, F_vals):
    x_t = x.transpose(2, 0, 1).reshape(NV, LAN)
    ewf = EW.reshape(-1)
    nsf = NS.reshape(-1)
    pad_f = NV_PAD * 6 - NV * 6
    pad_l = NV_PAD * 7 - NV * 7
    fcols = jnp.concatenate(
        [F_cols, (jnp.arange(pad_f, dtype=jnp.int32) * 797) % NF])
    fvals = jnp.concatenate([F_vals, jnp.zeros((pad_f,), jnp.float32)])
    lcols = jnp.concatenate(
        [L_cols, (jnp.arange(pad_l, dtype=jnp.int32) * 797) % NV])
    lvals = jnp.concatenate([L_vals, jnp.zeros((pad_l,), jnp.float32)])

    gf2 = _stage_a(x_t, G_cols, G_vals, ewf, nsf)
    if isinstance(gf2, (list, tuple)):
        gf2 = gf2[0]
    gout = _stage_b(gf2, x_t, fcols, fvals, lcols, lvals)
    if isinstance(gout, (list, tuple)):
        gout = gout[0]

    wk = coeffs.transpose(2, 1, 0)
    eye = jnp.eye(B, dtype=jnp.float32)
    bd = jax.vmap(lambda w: jnp.kron(eye, w))(wk)

    out_t = _stage_d(x_t, gout, bd)
    return out_t.reshape(NV, B, COUT).transpose(1, 2, 0)


# speedup vs baseline: 294.8617x; 1.0093x over previous
import functools

import jax
import jax.numpy as jnp
from jax import lax
from jax.experimental import pallas as pl
from jax.experimental.pallas import tpu as pltpu
from jax.experimental.pallas import tpu_sc as plsc

NV = 40962
NF = 81920
B = 4
CIN = 32
COUT = 32
LAN = B * CIN
NSEG = LAN // 16

NW = 32
CF = 32
FPT = NF // NW
A_CHUNKS = FPT // CF

CV = 16
VPT = 1312
NV_PAD = NW * VPT
B_CHUNKS = VPT // CV

_mesh = plsc.VectorSubcoreMesh(core_axis_name="c", subcore_axis_name="s")


def _wid():
    return lax.axis_index("s") * 2 + lax.axis_index("c")


@functools.partial(
    pl.kernel,
    mesh=_mesh,
    out_type=[
        jax.ShapeDtypeStruct((NF, 2 * LAN), jnp.float32),
    ],
    scratch_types=[
        pltpu.VMEM((2, 3, 3 * CF), jnp.int32),
        pltpu.VMEM((2, 3, 3 * CF + 16), jnp.float32),
        pltpu.VMEM((2, 3 * CF + 16), jnp.float32),
        pltpu.VMEM((2, 3 * CF + 16), jnp.float32),
        pltpu.VMEM((2, 9 * CF, LAN), jnp.float32),
        pltpu.VMEM((2, CF, 2 * LAN), jnp.float32),
        pltpu.SemaphoreType.DMA,
        pltpu.SemaphoreType.DMA,
        pltpu.SemaphoreType.DMA,
        pltpu.SemaphoreType.DMA,
        pltpu.SemaphoreType.DMA,
        pltpu.SemaphoreType.DMA,
    ],
)
def _stage_a(xt, gcols, gvals, ewf, nsf, gf2,
             idxb, gvb, ewb, nsb, rowsb, o2b,
             si0, si1, sg0, sg1, so0, so1):
    wid = _wid()
    f0 = wid * FPT
    sis = (si0, si1)
    sgs = (sg0, sg1)
    sos = (so0, so1)

    def start_idx(ch, b):
        fb = f0 + ch * CF
        for d in range(3):
            pltpu.async_copy(gcols.at[pl.ds(3 * d * NF + 3 * fb, 3 * CF)],
                             idxb.at[b, d], sis[b])

    def wait_idx(b):
        for d in range(3):
            pltpu.make_async_copy(gcols.at[pl.ds(0, 3 * CF)],
                                  idxb.at[b, d], sis[b]).wait()

    def start_gather(ch, b):
        fb = f0 + ch * CF
        for d in range(3):
            pltpu.async_copy(xt.at[idxb.at[b, d]],
                             rowsb.at[b, pl.ds(d * 3 * CF, 3 * CF)], sgs[b])
            pltpu.async_copy(gvals.at[pl.ds(3 * d * NF + 3 * fb, 3 * CF)],
                             gvb.at[b, d, pl.ds(0, 3 * CF)], sgs[b])
        pltpu.async_copy(ewf.at[pl.ds(3 * fb, 3 * CF)],
                         ewb.at[b, pl.ds(0, 3 * CF)], sgs[b])
        pltpu.async_copy(nsf.at[pl.ds(3 * fb, 3 * CF)],
                         nsb.at[b, pl.ds(0, 3 * CF)], sgs[b])

    def wait_gather(b):
        for d in range(3):
            pltpu.make_async_copy(xt.at[pl.ds(0, 3 * CF)],
                                  rowsb.at[b, pl.ds(d * 3 * CF, 3 * CF)],
                                  sgs[b]).wait()
            pltpu.make_async_copy(gvals.at[pl.ds(0, 3 * CF)],
                                  gvb.at[b, d, pl.ds(0, 3 * CF)], sgs[b]).wait()
        pltpu.make_async_copy(ewf.at[pl.ds(0, 3 * CF)],
                              ewb.at[b, pl.ds(0, 3 * CF)], sgs[b]).wait()
        pltpu.make_async_copy(nsf.at[pl.ds(0, 3 * CF)],
                              nsb.at[b, pl.ds(0, 3 * CF)], sgs[b]).wait()

    def start_out(ch, b):
        fb = f0 + ch * CF
        pltpu.async_copy(o2b.at[b], gf2.at[pl.ds(fb, CF)], sos[b])

    def wait_out(b):
        pltpu.make_async_copy(o2b.at[b], gf2.at[pl.ds(0, CF)], sos[b]).wait()

    def compute(b):
        def face(fl, inner):
            acc_e = [jnp.zeros((16,), jnp.float32) for _ in range(NSEG)]
            acc_n = [jnp.zeros((16,), jnp.float32) for _ in range(NSEG)]
            ew_vec = ewb[b, pl.ds(3 * fl, 16)]
            ns_vec = nsb[b, pl.ds(3 * fl, 16)]
            gv_vecs = [gvb[b, d, pl.ds(3 * fl, 16)] for d in range(3)]
            for d in range(3):
                we_d = ew_vec[d]
                wn_d = ns_vec[d]
                for j in range(3):
                    g = gv_vecs[d][j]
                    we = we_d * g
                    wn = wn_d * g
                    r = d * 3 * CF + 3 * fl + j
                    for s in range(NSEG):
                        seg = rowsb[b, r, pl.ds(16 * s, 16)]
                        acc_e[s] = acc_e[s] + we * seg
                        acc_n[s] = acc_n[s] + wn * seg
            for s in range(NSEG):
                o2b[b, fl, pl.ds(16 * s, 16)] = acc_e[s]
                o2b[b, fl, pl.ds(LAN + 16 * s, 16)] = acc_n[s]
            return inner

        lax.fori_loop(0, CF, face, 0)

    start_idx(0, 0)
    wait_idx(0)
    start_gather(0, 0)
    start_idx(1, 1)

    def body2(g, carry):
        for b in range(2):
            ch = g + b
            wait_gather(b)

            @pl.when(ch + 1 < A_CHUNKS)
            def _():
                wait_idx(1 - b)
                start_gather(ch + 1, 1 - b)

            @pl.when(ch + 2 < A_CHUNKS)
            def _():
                start_idx(ch + 2, b)

            @pl.when(ch >= 2)
            def _():
                wait_out(b)

            compute(b)
            start_out(ch, b)
        return carry

    lax.fori_loop(0, A_CHUNKS // 2, lambda i, c: body2(2 * i, c), 0)
    wait_out(0)
    wait_out(1)


@functools.partial(
    pl.kernel,
    mesh=_mesh,
    out_type=[
        jax.ShapeDtypeStruct((NV_PAD, 3 * LAN), jnp.float32),
    ],
    scratch_types=[
        pltpu.VMEM((2, 6 * CV), jnp.int32),
        pltpu.VMEM((2, 6 * CV + 16), jnp.float32),
        pltpu.VMEM((2, 7 * CV), jnp.int32),
        pltpu.VMEM((2, 7 * CV + 16), jnp.float32),
        pltpu.VMEM((2, 6 * CV, 2 * LAN), jnp.float32),
        pltpu.VMEM((2, 7 * CV, LAN), jnp.float32),
        pltpu.VMEM((2, CV, 3 * LAN), jnp.float32),
        pltpu.SemaphoreType.DMA,
        pltpu.SemaphoreType.DMA,
        pltpu.SemaphoreType.DMA,
        pltpu.SemaphoreType.DMA,
        pltpu.SemaphoreType.DMA,
        pltpu.SemaphoreType.DMA,
    ],
)
def _stage_b(gf2, xt, fcols, fvals, lcols, lvals,
             gout,
             fib, fvb, lib, lvb, r2b, rlb, o3b,
             si0, si1, sg0, sg1, so0, so1):
    wid = _wid()
    v0 = wid * VPT
    sis = (si0, si1)
    sgs = (sg0, sg1)
    sos = (so0, so1)

    def start_idx(ch, b):
        vb = v0 + ch * CV
        pltpu.async_copy(fcols.at[pl.ds(6 * vb, 6 * CV)], fib.at[b], sis[b])
        pltpu.async_copy(lcols.at[pl.ds(7 * vb, 7 * CV)], lib.at[b], sis[b])

    def wait_idx(b):
        pltpu.make_async_copy(fcols.at[pl.ds(0, 6 * CV)], fib.at[b], sis[b]).wait()
        pltpu.make_async_copy(lcols.at[pl.ds(0, 7 * CV)], lib.at[b], sis[b]).wait()

    def start_gather(ch, b):
        vb = v0 + ch * CV
        pltpu.async_copy(gf2.at[fib.at[b]], r2b.at[b], sgs[b])
        pltpu.async_copy(xt.at[lib.at[b]], rlb.at[b], sgs[b])
        pltpu.async_copy(fvals.at[pl.ds(6 * vb, 6 * CV)],
                         fvb.at[b, pl.ds(0, 6 * CV)], sgs[b])
        pltpu.async_copy(lvals.at[pl.ds(7 * vb, 7 * CV)],
                         lvb.at[b, pl.ds(0, 7 * CV)], sgs[b])

    def wait_gather(b):
        pltpu.make_async_copy(gf2.at[pl.ds(0, 6 * CV)], r2b.at[b], sgs[b]).wait()
        pltpu.make_async_copy(xt.at[pl.ds(0, 7 * CV)], rlb.at[b], sgs[b]).wait()
        pltpu.make_async_copy(fvals.at[pl.ds(0, 6 * CV)],
                              fvb.at[b, pl.ds(0, 6 * CV)], sgs[b]).wait()
        pltpu.make_async_copy(lvals.at[pl.ds(0, 7 * CV)],
                              lvb.at[b, pl.ds(0, 7 * CV)], sgs[b]).wait()

    def start_out(ch, b):
        vb = v0 + ch * CV
        pltpu.async_copy(o3b.at[b], gout.at[pl.ds(vb, CV)], sos[b])

    def wait_out(b):
        pltpu.make_async_copy(o3b.at[b], gout.at[pl.ds(0, CV)], sos[b]).wait()

    def compute(b):
        def vert(vl, inner):
            acc_e = [jnp.zeros((16,), jnp.float32) for _ in range(NSEG)]
            acc_n = [jnp.zeros((16,), jnp.float32) for _ in range(NSEG)]
            acc_l = [jnp.zeros((16,), jnp.float32) for _ in range(NSEG)]
            fv_vec = fvb[b, pl.ds(6 * vl, 16)]
            lv_vec = lvb[b, pl.ds(7 * vl, 16)]
            for j in range(6):
                r = 6 * vl + j
                w = fv_vec[j]
                for s in range(NSEG):
                    acc_e[s] = acc_e[s] + w * r2b[b, r, pl.ds(16 * s, 16)]
                    acc_n[s] = acc_n[s] + w * r2b[b, r, pl.ds(LAN + 16 * s, 16)]
            for j in range(7):
                r = 7 * vl + j
                w = lv_vec[j]
                for s in range(NSEG):
                    acc_l[s] = acc_l[s] + w * rlb[b, r, pl.ds(16 * s, 16)]
            for s in range(NSEG):
                o3b[b, vl, pl.ds(16 * s, 16)] = acc_e[s]
                o3b[b, vl, pl.ds(LAN + 16 * s, 16)] = acc_n[s]
                o3b[b, vl, pl.ds(2 * LAN + 16 * s, 16)] = acc_l[s]
            return inner

        lax.fori_loop(0, CV, vert, 0)

    start_idx(0, 0)
    wait_idx(0)
    start_gather(0, 0)
    start_idx(1, 1)

    def body2(g, carry):
        for b in range(2):
            ch = g + b
            wait_gather(b)

            @pl.when(ch + 1 < B_CHUNKS)
            def _():
                wait_idx(1 - b)
                start_gather(ch + 1, 1 - b)

            @pl.when(ch + 2 < B_CHUNKS)
            def _():
                start_idx(ch + 2, b)

            @pl.when(ch >= 2)
            def _():
                wait_out(b)

            compute(b)
            start_out(ch, b)
        return carry

    lax.fori_loop(0, B_CHUNKS // 2, lambda i, c: body2(2 * i, c), 0)
    wait_out(0)
    wait_out(1)


TR = 512
GRID_D = (NV + TR - 1) // TR


def _stage_d_body(x_r, gv_r, w0_r, m_r, out_r):
    gv = gv_r[...]
    w0 = w0_r[...]
    for b in range(B):
        acc = lax.dot_general(w0, x_r[b], (((1,), (0,)), ((), ())),
                              preferred_element_type=jnp.float32)
        acc = acc + lax.dot_general(m_r[b], gv, (((0,), (1,)), ((), ())),
                                    preferred_element_type=jnp.float32)
        out_r[b] = acc


def _stage_d(x, gout, w0, m):
    return pl.pallas_call(
        _stage_d_body,
        grid=(GRID_D,),
        in_specs=[pl.BlockSpec((B, CIN, TR), lambda i: (0, 0, i)),
                  pl.BlockSpec((TR, 3 * LAN), lambda i: (i, 0)),
                  pl.BlockSpec((COUT, CIN), lambda i: (0, 0)),
                  pl.BlockSpec((B, 3 * LAN, COUT), lambda i: (0, 0, 0))],
        out_specs=pl.BlockSpec((B, COUT, TR), lambda i: (0, 0, i)),
        out_shape=jax.ShapeDtypeStruct((B, COUT, NV), jnp.float32),
    )(x, gout, w0, m)


def kernel(x, coeffs, G_rows, G_cols, G_vals, EW, NS,
           L_rows, L_cols, L_vals, F_rows, F_cols, F_vals):
    x_t = x.transpose(2, 0, 1).reshape(NV, LAN)
    ewf = EW.reshape(-1)
    nsf = NS.reshape(-1)
    pad_f = NV_PAD * 6 - NV * 6
    pad_l = NV_PAD * 7 - NV * 7
    fcols = jnp.concatenate(
        [F_cols, (jnp.arange(pad_f, dtype=jnp.int32) * 797) % NF])
    fvals = jnp.concatenate([F_vals, jnp.zeros((pad_f,), jnp.float32)])
    lcols = jnp.concatenate(
        [L_cols, (jnp.arange(pad_l, dtype=jnp.int32) * 797) % NV])
    lvals = jnp.concatenate([L_vals, jnp.zeros((pad_l,), jnp.float32)])

    gf2 = _stage_a(x_t, G_cols, G_vals, ewf, nsf)
    if isinstance(gf2, (list, tuple)):
        gf2 = gf2[0]
    gout = _stage_b(gf2, x_t, fcols, fvals, lcols, lvals)
    if isinstance(gout, (list, tuple)):
        gout = gout[0]

    wk = coeffs.transpose(2, 1, 0)
    bi = jnp.arange(B)[:, None, None]
    ki = jnp.arange(3)[None, :, None]
    ii = jnp.arange(CIN)[None, None, :]
    m = jnp.zeros((B, 3 * LAN, COUT), jnp.float32)
    m = m.at[bi, ki * LAN + bi * CIN + ii, :].set(wk[1:][ki, ii])
    w0 = coeffs[:, :, 0]

    return _stage_d(x, gout, w0, m)


# speedup vs baseline: 299.3623x; 1.0153x over previous
import functools

import jax
import jax.numpy as jnp
from jax import lax
from jax.experimental import pallas as pl
from jax.experimental.pallas import tpu as pltpu
from jax.experimental.pallas import tpu_sc as plsc

NV = 40962
NF = 81920
B = 4
CIN = 32
COUT = 32
LAN = B * CIN
NSEG = LAN // 16

NW = 32
CF = 32
FPT = NF // NW
A_CHUNKS = FPT // CF

CV = 16
VPT = 1312
NV_PAD = NW * VPT
B_CHUNKS = VPT // CV

_mesh = plsc.VectorSubcoreMesh(core_axis_name="c", subcore_axis_name="s")


def _wid():
    return lax.axis_index("s") * 2 + lax.axis_index("c")


@functools.partial(
    pl.kernel,
    mesh=_mesh,
    out_type=[
        jax.ShapeDtypeStruct((NF, 2 * LAN), jnp.float32),
    ],
    scratch_types=[
        pltpu.VMEM((2, 3, 3 * CF), jnp.int32),
        pltpu.VMEM((2, 3, 3 * CF + 16), jnp.float32),
        pltpu.VMEM((2, 3 * CF + 16), jnp.float32),
        pltpu.VMEM((2, 3 * CF + 16), jnp.float32),
        pltpu.VMEM((2, 9 * CF, LAN), jnp.float32),
        pltpu.VMEM((2, CF, 2 * LAN), jnp.float32),
        pltpu.SemaphoreType.DMA,
        pltpu.SemaphoreType.DMA,
        pltpu.SemaphoreType.DMA,
        pltpu.SemaphoreType.DMA,
        pltpu.SemaphoreType.DMA,
        pltpu.SemaphoreType.DMA,
    ],
)
def _stage_a(xt, gcols, gvals, ewf, nsf, gf2,
             idxb, gvb, ewb, nsb, rowsb, o2b,
             si0, si1, sg0, sg1, so0, so1):
    wid = _wid()
    f0 = wid * FPT
    sis = (si0, si1)
    sgs = (sg0, sg1)
    sos = (so0, so1)

    def start_idx(ch, b):
        fb = f0 + ch * CF
        for d in range(3):
            pltpu.async_copy(gcols.at[pl.ds(3 * d * NF + 3 * fb, 3 * CF)],
                             idxb.at[b, d], sis[b])

    def wait_idx(b):
        for d in range(3):
            pltpu.make_async_copy(gcols.at[pl.ds(0, 3 * CF)],
                                  idxb.at[b, d], sis[b]).wait()

    def start_gather(ch, b):
        fb = f0 + ch * CF
        for d in range(3):
            pltpu.async_copy(xt.at[idxb.at[b, d]],
                             rowsb.at[b, pl.ds(d * 3 * CF, 3 * CF)], sgs[b])
            pltpu.async_copy(gvals.at[pl.ds(3 * d * NF + 3 * fb, 3 * CF)],
                             gvb.at[b, d, pl.ds(0, 3 * CF)], sgs[b])
        pltpu.async_copy(ewf.at[pl.ds(3 * fb, 3 * CF)],
                         ewb.at[b, pl.ds(0, 3 * CF)], sgs[b])
        pltpu.async_copy(nsf.at[pl.ds(3 * fb, 3 * CF)],
                         nsb.at[b, pl.ds(0, 3 * CF)], sgs[b])

    def wait_gather(b):
        for d in range(3):
            pltpu.make_async_copy(xt.at[pl.ds(0, 3 * CF)],
                                  rowsb.at[b, pl.ds(d * 3 * CF, 3 * CF)],
                                  sgs[b]).wait()
            pltpu.make_async_copy(gvals.at[pl.ds(0, 3 * CF)],
                                  gvb.at[b, d, pl.ds(0, 3 * CF)], sgs[b]).wait()
        pltpu.make_async_copy(ewf.at[pl.ds(0, 3 * CF)],
                              ewb.at[b, pl.ds(0, 3 * CF)], sgs[b]).wait()
        pltpu.make_async_copy(nsf.at[pl.ds(0, 3 * CF)],
                              nsb.at[b, pl.ds(0, 3 * CF)], sgs[b]).wait()

    def start_out(ch, b):
        fb = f0 + ch * CF
        pltpu.async_copy(o2b.at[b], gf2.at[pl.ds(fb, CF)], sos[b])

    def wait_out(b):
        pltpu.make_async_copy(o2b.at[b], gf2.at[pl.ds(0, CF)], sos[b]).wait()

    def compute(b):
        def face(fl, inner):
            acc_e = [jnp.zeros((16,), jnp.float32) for _ in range(NSEG)]
            acc_n = [jnp.zeros((16,), jnp.float32) for _ in range(NSEG)]
            ew_vec = ewb[b, pl.ds(3 * fl, 16)]
            ns_vec = nsb[b, pl.ds(3 * fl, 16)]
            gv_vecs = [gvb[b, d, pl.ds(3 * fl, 16)] for d in range(3)]
            for d in range(3):
                we_d = ew_vec[d]
                wn_d = ns_vec[d]
                for j in range(3):
                    g = gv_vecs[d][j]
                    we = we_d * g
                    wn = wn_d * g
                    r = d * 3 * CF + 3 * fl + j
                    for s in range(NSEG):
                        seg = rowsb[b, r, pl.ds(16 * s, 16)]
                        acc_e[s] = acc_e[s] + we * seg
                        acc_n[s] = acc_n[s] + wn * seg
            for s in range(NSEG):
                o2b[b, fl, pl.ds(16 * s, 16)] = acc_e[s]
                o2b[b, fl, pl.ds(LAN + 16 * s, 16)] = acc_n[s]
            return inner

        lax.fori_loop(0, CF, face, 0)

    start_idx(0, 0)
    wait_idx(0)
    start_gather(0, 0)
    start_idx(1, 1)

    def body2(g, carry):
        for b in range(2):
            ch = g + b
            wait_gather(b)

            @pl.when(ch + 1 < A_CHUNKS)
            def _():
                wait_idx(1 - b)
                start_gather(ch + 1, 1 - b)

            @pl.when(ch + 2 < A_CHUNKS)
            def _():
                start_idx(ch + 2, b)

            @pl.when(ch >= 2)
            def _():
                wait_out(b)

            compute(b)
            start_out(ch, b)
        return carry

    lax.fori_loop(0, A_CHUNKS // 2, lambda i, c: body2(2 * i, c), 0)
    wait_out(0)
    wait_out(1)


@functools.partial(
    pl.kernel,
    mesh=_mesh,
    out_type=[
        jax.ShapeDtypeStruct((NV_PAD, 3 * LAN), jnp.float32),
    ],
    scratch_types=[
        pltpu.VMEM((2, 6 * CV), jnp.int32),
        pltpu.VMEM((2, 6 * CV + 16), jnp.float32),
        pltpu.VMEM((2, 7 * CV), jnp.int32),
        pltpu.VMEM((2, 7 * CV + 16), jnp.float32),
        pltpu.VMEM((2, 6 * CV, 2 * LAN), jnp.float32),
        pltpu.VMEM((2, 7 * CV, LAN), jnp.float32),
        pltpu.VMEM((2, CV, 3 * LAN), jnp.float32),
        pltpu.SemaphoreType.DMA,
        pltpu.SemaphoreType.DMA,
        pltpu.SemaphoreType.DMA,
        pltpu.SemaphoreType.DMA,
        pltpu.SemaphoreType.DMA,
        pltpu.SemaphoreType.DMA,
    ],
)
def _stage_b(gf2, xt, fcols, fvals, lcols, lvals,
             gout,
             fib, fvb, lib, lvb, r2b, rlb, o3b,
             si0, si1, sg0, sg1, so0, so1):
    wid = _wid()
    v0 = wid * VPT
    sis = (si0, si1)
    sgs = (sg0, sg1)
    sos = (so0, so1)

    def start_idx(ch, b):
        vb = v0 + ch * CV
        pltpu.async_copy(fcols.at[pl.ds(6 * vb, 6 * CV)], fib.at[b], sis[b])
        pltpu.async_copy(lcols.at[pl.ds(7 * vb, 7 * CV)], lib.at[b], sis[b])

    def wait_idx(b):
        pltpu.make_async_copy(fcols.at[pl.ds(0, 6 * CV)], fib.at[b], sis[b]).wait()
        pltpu.make_async_copy(lcols.at[pl.ds(0, 7 * CV)], lib.at[b], sis[b]).wait()

    def start_gather(ch, b):
        vb = v0 + ch * CV
        pltpu.async_copy(gf2.at[fib.at[b]], r2b.at[b], sgs[b])
        pltpu.async_copy(xt.at[lib.at[b]], rlb.at[b], sgs[b])
        pltpu.async_copy(fvals.at[pl.ds(6 * vb, 6 * CV)],
                         fvb.at[b, pl.ds(0, 6 * CV)], sgs[b])
        pltpu.async_copy(lvals.at[pl.ds(7 * vb, 7 * CV)],
                         lvb.at[b, pl.ds(0, 7 * CV)], sgs[b])

    def wait_gather(b):
        pltpu.make_async_copy(gf2.at[pl.ds(0, 6 * CV)], r2b.at[b], sgs[b]).wait()
        pltpu.make_async_copy(xt.at[pl.ds(0, 7 * CV)], rlb.at[b], sgs[b]).wait()
        pltpu.make_async_copy(fvals.at[pl.ds(0, 6 * CV)],
                              fvb.at[b, pl.ds(0, 6 * CV)], sgs[b]).wait()
        pltpu.make_async_copy(lvals.at[pl.ds(0, 7 * CV)],
                              lvb.at[b, pl.ds(0, 7 * CV)], sgs[b]).wait()

    def start_out(ch, b):
        vb = v0 + ch * CV
        pltpu.async_copy(o3b.at[b], gout.at[pl.ds(vb, CV)], sos[b])

    def wait_out(b):
        pltpu.make_async_copy(o3b.at[b], gout.at[pl.ds(0, CV)], sos[b]).wait()

    def compute(b):
        def vert(vl, inner):
            acc_e = [jnp.zeros((16,), jnp.float32) for _ in range(NSEG)]
            acc_n = [jnp.zeros((16,), jnp.float32) for _ in range(NSEG)]
            acc_l = [jnp.zeros((16,), jnp.float32) for _ in range(NSEG)]
            fv_vec = fvb[b, pl.ds(6 * vl, 16)]
            lv_vec = lvb[b, pl.ds(7 * vl, 16)]
            for j in range(6):
                r = 6 * vl + j
                w = fv_vec[j]
                for s in range(NSEG):
                    acc_e[s] = acc_e[s] + w * r2b[b, r, pl.ds(16 * s, 16)]
                    acc_n[s] = acc_n[s] + w * r2b[b, r, pl.ds(LAN + 16 * s, 16)]
            for j in range(7):
                r = 7 * vl + j
                w = lv_vec[j]
                for s in range(NSEG):
                    acc_l[s] = acc_l[s] + w * rlb[b, r, pl.ds(16 * s, 16)]
            for s in range(NSEG):
                o3b[b, vl, pl.ds(16 * s, 16)] = acc_e[s]
                o3b[b, vl, pl.ds(LAN + 16 * s, 16)] = acc_n[s]
                o3b[b, vl, pl.ds(2 * LAN + 16 * s, 16)] = acc_l[s]
            return inner

        lax.fori_loop(0, CV, vert, 0)

    start_idx(0, 0)
    wait_idx(0)
    start_gather(0, 0)
    start_idx(1, 1)

    def body2(g, carry):
        for b in range(2):
            ch = g + b
            wait_gather(b)

            @pl.when(ch + 1 < B_CHUNKS)
            def _():
                wait_idx(1 - b)
                start_gather(ch + 1, 1 - b)

            @pl.when(ch + 2 < B_CHUNKS)
            def _():
                start_idx(ch + 2, b)

            @pl.when(ch >= 2)
            def _():
                wait_out(b)

            compute(b)
            start_out(ch, b)
        return carry

    lax.fori_loop(0, B_CHUNKS // 2, lambda i, c: body2(2 * i, c), 0)
    wait_out(0)
    wait_out(1)


TRT = 1024
GRID_T = (NV + TRT - 1) // TRT


def _tr_body(x_r, out_r):
    v = x_r[...].reshape(LAN, TRT)
    out_r[...] = v.T


def _transpose_x(x):
    return pl.pallas_call(
        _tr_body,
        grid=(GRID_T,),
        in_specs=[pl.BlockSpec((B, CIN, TRT), lambda i: (0, 0, i))],
        out_specs=pl.BlockSpec((TRT, LAN), lambda i: (i, 0)),
        out_shape=jax.ShapeDtypeStruct((NV, LAN), jnp.float32),
    )(x)


TR = 1024
GRID_D = (NV + TR - 1) // TR


def _stage_d_body(x_r, gv_r, w0_r, m_r, out_r):
    gv = gv_r[...]
    w0 = w0_r[...]
    for b in range(B):
        acc = lax.dot_general(w0, x_r[b], (((1,), (0,)), ((), ())),
                              preferred_element_type=jnp.float32)
        acc = acc + lax.dot_general(m_r[b], gv, (((0,), (1,)), ((), ())),
                                    preferred_element_type=jnp.float32)
        out_r[b] = acc


def _stage_d(x, gout, w0, m):
    return pl.pallas_call(
        _stage_d_body,
        grid=(GRID_D,),
        in_specs=[pl.BlockSpec((B, CIN, TR), lambda i: (0, 0, i)),
                  pl.BlockSpec((TR, 3 * LAN), lambda i: (i, 0)),
                  pl.BlockSpec((COUT, CIN), lambda i: (0, 0)),
                  pl.BlockSpec((B, 3 * LAN, COUT), lambda i: (0, 0, 0))],
        out_specs=pl.BlockSpec((B, COUT, TR), lambda i: (0, 0, i)),
        out_shape=jax.ShapeDtypeStruct((B, COUT, NV), jnp.float32),
    )(x, gout, w0, m)


def kernel(x, coeffs, G_rows, G_cols, G_vals, EW, NS,
           L_rows, L_cols, L_vals, F_rows, F_cols, F_vals):
    x_t = _transpose_x(x)
    ewf = EW.reshape(-1)
    nsf = NS.reshape(-1)
    pad_f = NV_PAD * 6 - NV * 6
    pad_l = NV_PAD * 7 - NV * 7
    fcols = jnp.concatenate(
        [F_cols, (jnp.arange(pad_f, dtype=jnp.int32) * 797) % NF])
    fvals = jnp.concatenate([F_vals, jnp.zeros((pad_f,), jnp.float32)])
    lcols = jnp.concatenate(
        [L_cols, (jnp.arange(pad_l, dtype=jnp.int32) * 797) % NV])
    lvals = jnp.concatenate([L_vals, jnp.zeros((pad_l,), jnp.float32)])

    gf2 = _stage_a(x_t, G_cols, G_vals, ewf, nsf)
    if isinstance(gf2, (list, tuple)):
        gf2 = gf2[0]
    gout = _stage_b(gf2, x_t, fcols, fvals, lcols, lvals)
    if isinstance(gout, (list, tuple)):
        gout = gout[0]

    wk = coeffs.transpose(2, 1, 0)
    bi = jnp.arange(B)[:, None, None]
    ki = jnp.arange(3)[None, :, None]
    ii = jnp.arange(CIN)[None, None, :]
    m = jnp.zeros((B, 3 * LAN, COUT), jnp.float32)
    m = m.at[bi, ki * LAN + bi * CIN + ii, :].set(wk[1:][ki, ii])
    w0 = coeffs[:, :, 0]

    return _stage_d(x, gout, w0, m)
